# baseline (speedup 1.0000x reference)
import functools

import jax
import jax.numpy as jnp
from jax import lax
from jax.experimental import pallas as pl
from jax.experimental.pallas import tpu as pltpu
from jax.experimental.pallas import tpu_sc as plsc

NUM_EXPERTS = 64
TOP_K = 2
HIDDEN = 1024
FF = 768
SC_EXPERTS = 16


def _moe_kernel(x_ref, rw_ref, wg_ref, wu_ref, wd_ref, out_ref, wn_ref, idx_ref):
    e = pl.program_id(0)
    x = x_ref[...]

    @pl.when(e == 0)
    def _router():
        logits = jnp.dot(x, rw_ref[...], preferred_element_type=jnp.float32)
        probs = jax.nn.softmax(logits, axis=-1)
        T, E = probs.shape
        col = jax.lax.broadcasted_iota(jnp.int32, (T, E), 1)
        w1 = jnp.max(probs, axis=-1, keepdims=True)
        i1 = jnp.argmax(probs, axis=-1).reshape(T, 1)
        masked = jnp.where(col == i1, -1.0, probs)
        w2 = jnp.max(masked, axis=-1, keepdims=True)
        i2 = jnp.argmax(masked, axis=-1).reshape(T, 1)
        s = w1 + w2
        wn_ref[:, 0:1] = w1 / s
        wn_ref[:, 1:2] = w2 / s
        idx_ref[:, 0:1] = i1
        idx_ref[:, 1:2] = i2
        out_ref[...] = jnp.zeros_like(out_ref)

    wg = wg_ref[0]
    wu = wu_ref[0]
    wd = wd_ref[0]
    g = jnp.dot(x, wg, preferred_element_type=jnp.float32)
    u = jnp.dot(x, wu, preferred_element_type=jnp.float32)
    h = (g * jax.nn.sigmoid(g)) * u
    w_e = (
        jnp.where(idx_ref[:, 0:1] == e, wn_ref[:, 0:1], 0.0)
        + jnp.where(idx_ref[:, 1:2] == e, wn_ref[:, 1:2], 0.0)
    )
    out_ref[...] += jnp.dot(w_e * h, wd, preferred_element_type=jnp.float32)


def _tc_moe(x, router_weight, gate_proj, up_proj, down_proj, t, d):
    return pl.pallas_call(
        _moe_kernel,
        grid=(NUM_EXPERTS,),
        in_specs=[
            pl.BlockSpec((t, d), lambda e: (0, 0)),
            pl.BlockSpec((d, NUM_EXPERTS), lambda e: (0, 0)),
            pl.BlockSpec((1, HIDDEN, FF), lambda e: (e, 0, 0)),
            pl.BlockSpec((1, HIDDEN, FF), lambda e: (e, 0, 0)),
            pl.BlockSpec((1, FF, HIDDEN), lambda e: (e, 0, 0)),
        ],
        out_specs=pl.BlockSpec((t, d), lambda e: (0, 0)),
        out_shape=jax.ShapeDtypeStruct((t, d), jnp.float32),
        scratch_shapes=[
            pltpu.VMEM((t, TOP_K), jnp.float32),
            pltpu.VMEM((t, TOP_K), jnp.int32),
        ],
        compiler_params=pltpu.CompilerParams(
            dimension_semantics=("arbitrary",),
        ),
    )(x, router_weight, gate_proj, up_proj, down_proj)


def _sc_probe(gate_proj, up_proj, down_proj):
    mesh = plsc.VectorSubcoreMesh(core_axis_name="c", subcore_axis_name="s")

    @functools.partial(
        pl.kernel,
        mesh=mesh,
        out_type=jax.ShapeDtypeStruct((32, 16), jnp.float32),
        scratch_types=[
            pltpu.VMEM((64, FF), jnp.float32),
            pltpu.VMEM((48, HIDDEN), jnp.float32),
            pltpu.VMEM((16,), jnp.float32),
        ],
    )
    def body(gp_hbm, up_hbm, dp_hbm, out_hbm, buf_a, buf_b, stage):
        wid = lax.axis_index("s") * 2 + lax.axis_index("c")
        e = (NUM_EXPERTS - SC_EXPERTS) + jnp.remainder(wid, SC_EXPERTS)
        half = wid // SC_EXPERTS
        row0 = half * (HIDDEN // 2)
        for r in range(8):
            pltpu.sync_copy(gp_hbm.at[e, pl.ds(row0 + r * 64, 64)], buf_a)
            pltpu.sync_copy(up_hbm.at[e, pl.ds(row0 + r * 64, 64)], buf_a)
        drow0 = half * (FF // 2)
        for r in range(8):
            pltpu.sync_copy(dp_hbm.at[e, pl.ds(drow0 + r * 48, 48)], buf_b)
        stage[...] = buf_a[0, pl.ds(0, 16)] + buf_b[0, pl.ds(0, 16)]
        pltpu.sync_copy(stage, out_hbm.at[wid])

    return body(gate_proj, up_proj, down_proj)


@jax.jit
def kernel(hidden_states, router_weight, gate_proj, up_proj, down_proj):
    b, s, d = hidden_states.shape
    x = hidden_states.reshape(-1, d)
    t = x.shape[0]
    sc = _sc_probe(gate_proj, up_proj, down_proj)
    out = _tc_moe(x, router_weight, gate_proj, up_proj, down_proj, t, d)
    out = out + 0.0 * jnp.sum(sc)
    return out.reshape(b, s, d)


# speedup vs baseline: 1.2942x; 1.2942x over previous
import functools

import jax
import jax.numpy as jnp
from jax.experimental import pallas as pl
from jax.experimental.pallas import tpu as pltpu

NUM_EXPERTS = 64
TOP_K = 2
HIDDEN = 1024
FF = 768


def _moe_kernel(x_ref, rw_ref, wg_ref, wu_ref, wd_ref, out_ref, wn_ref, idx_ref):
    e = pl.program_id(0)
    x = x_ref[...]

    @pl.when(e == 0)
    def _router():
        logits = jnp.dot(x, rw_ref[...], preferred_element_type=jnp.float32)
        probs = jax.nn.softmax(logits, axis=-1)
        T, E = probs.shape
        col = jax.lax.broadcasted_iota(jnp.int32, (T, E), 1)
        w1 = jnp.max(probs, axis=-1, keepdims=True)
        i1 = jnp.argmax(probs, axis=-1).reshape(T, 1)
        masked = jnp.where(col == i1, -1.0, probs)
        w2 = jnp.max(masked, axis=-1, keepdims=True)
        i2 = jnp.argmax(masked, axis=-1).reshape(T, 1)
        s = w1 + w2
        wn_ref[:, 0:1] = w1 / s
        wn_ref[:, 1:2] = w2 / s
        idx_ref[:, 0:1] = i1
        idx_ref[:, 1:2] = i2
        out_ref[...] = jnp.zeros_like(out_ref)

    wg = wg_ref[0]
    wu = wu_ref[0]
    wd = wd_ref[0]
    g = jnp.dot(x, wg, preferred_element_type=jnp.float32)
    u = jnp.dot(x, wu, preferred_element_type=jnp.float32)
    h = (g * jax.nn.sigmoid(g)) * u
    w_e = (
        jnp.where(idx_ref[:, 0:1] == e, wn_ref[:, 0:1], 0.0)
        + jnp.where(idx_ref[:, 1:2] == e, wn_ref[:, 1:2], 0.0)
    )
    out_ref[...] += jnp.dot(w_e * h, wd, preferred_element_type=jnp.float32)


@functools.partial(jax.jit, static_argnames=("interpret",))
def kernel(hidden_states, router_weight, gate_proj, up_proj, down_proj,
           interpret=False):
    b, s, d = hidden_states.shape
    x = hidden_states.reshape(-1, d)
    t = x.shape[0]
    out = pl.pallas_call(
        _moe_kernel,
        grid=(NUM_EXPERTS,),
        in_specs=[
            pl.BlockSpec((t, d), lambda e: (0, 0)),
            pl.BlockSpec((d, NUM_EXPERTS), lambda e: (0, 0)),
            pl.BlockSpec((1, HIDDEN, FF), lambda e: (e, 0, 0)),
            pl.BlockSpec((1, HIDDEN, FF), lambda e: (e, 0, 0)),
            pl.BlockSpec((1, FF, HIDDEN), lambda e: (e, 0, 0)),
        ],
        out_specs=pl.BlockSpec((t, d), lambda e: (0, 0)),
        out_shape=jax.ShapeDtypeStruct((t, d), jnp.float32),
        scratch_shapes=[
            pltpu.VMEM((t, TOP_K), jnp.float32),
            pltpu.VMEM((t, TOP_K), jnp.int32),
        ],
        compiler_params=pltpu.CompilerParams(
            dimension_semantics=("arbitrary",),
        ),
        interpret=interpret,
    )(x, router_weight, gate_proj, up_proj, down_proj)
    return out.reshape(b, s, d)


# speedup vs baseline: 1.3354x; 1.0318x over previous
import functools

import jax
import jax.numpy as jnp
from jax.experimental import pallas as pl
from jax.experimental.pallas import tpu as pltpu

NUM_EXPERTS = 64
TOP_K = 2
HIDDEN = 1024
FF = 768
FF_CHUNK = 768


def _moe_kernel(x_ref, rw_ref, wg_ref, wu_ref, wd_ref, out_ref, wn_ref, idx_ref):
    e = pl.program_id(0)
    j = pl.program_id(1)
    x = x_ref[...]

    @pl.when((e == 0) & (j == 0))
    def _router():
        logits = jnp.dot(x, rw_ref[...], preferred_element_type=jnp.float32)
        probs = jax.nn.softmax(logits, axis=-1)
        T, E = probs.shape
        col = jax.lax.broadcasted_iota(jnp.int32, (T, E), 1)
        w1 = jnp.max(probs, axis=-1, keepdims=True)
        i1 = jnp.argmax(probs, axis=-1).reshape(T, 1)
        masked = jnp.where(col == i1, -1.0, probs)
        w2 = jnp.max(masked, axis=-1, keepdims=True)
        i2 = jnp.argmax(masked, axis=-1).reshape(T, 1)
        s = w1 + w2
        wn_ref[:, 0:1] = w1 / s
        wn_ref[:, 1:2] = w2 / s
        idx_ref[:, 0:1] = i1
        idx_ref[:, 1:2] = i2
        out_ref[...] = jnp.zeros_like(out_ref)

    wg = wg_ref[0]
    wu = wu_ref[0]
    wd = wd_ref[0]
    g = jnp.dot(x, wg, preferred_element_type=jnp.float32)
    u = jnp.dot(x, wu, preferred_element_type=jnp.float32)
    h = (g * jax.nn.sigmoid(g)) * u
    w_e = (
        jnp.where(idx_ref[:, 0:1] == e, wn_ref[:, 0:1], 0.0)
        + jnp.where(idx_ref[:, 1:2] == e, wn_ref[:, 1:2], 0.0)
    )
    y = jnp.dot(w_e * h, wd, preferred_element_type=jnp.float32)
    out_ref[...] += y


@functools.partial(jax.jit, static_argnames=("interpret",))
def kernel(hidden_states, router_weight, gate_proj, up_proj, down_proj,
           interpret=False):
    b, s, d = hidden_states.shape
    x = hidden_states.reshape(-1, d)
    t = x.shape[0]
    n_chunks = FF // FF_CHUNK
    out = pl.pallas_call(
        _moe_kernel,
        grid=(NUM_EXPERTS, n_chunks),
        in_specs=[
            pl.BlockSpec((t, d), lambda e, j: (0, 0)),
            pl.BlockSpec((d, NUM_EXPERTS), lambda e, j: (0, 0)),
            pl.BlockSpec((1, HIDDEN, FF_CHUNK), lambda e, j: (e, 0, j)),
            pl.BlockSpec((1, HIDDEN, FF_CHUNK), lambda e, j: (e, 0, j)),
            pl.BlockSpec((1, FF_CHUNK, HIDDEN), lambda e, j: (e, j, 0)),
        ],
        out_specs=pl.BlockSpec((t, d), lambda e, j: (0, 0)),
        out_shape=jax.ShapeDtypeStruct((t, d), jnp.float32),
        scratch_shapes=[
            pltpu.VMEM((t, TOP_K), jnp.float32),
            pltpu.VMEM((t, TOP_K), jnp.int32),
        ],
        compiler_params=pltpu.CompilerParams(
            dimension_semantics=("arbitrary", "arbitrary"),
        ),
        interpret=interpret,
    )(x, router_weight, gate_proj, up_proj, down_proj)
    return out.reshape(b, s, d)
